# Initial kernel scaffold; baseline (speedup 1.0000x reference)
import jax
import jax.numpy as jnp
from jax.experimental import pallas as pl


def kernel(x, edge_index, batch, W1, a_src1, a_dst1, b1, gamma1, beta1, W2, a_src2, a_dst2, b2, Wc1, bc1, Wc2, bc2):
    raise NotImplementedError("write your pallas kernel here")


# speedup vs baseline: 1.0729x; 1.0729x over previous
import functools

import jax
import jax.numpy as jnp
from jax.experimental import pallas as pl

N = 10000
E = 320000
D = 128
H = 8
C = 128
B = 64
OUT = 10


def _mm_body(x_ref, w_ref, o_ref):
    o_ref[...] = jnp.dot(x_ref[...], w_ref[...], preferred_element_type=jnp.float32)


def _matmul(x, w, bn=400):
    n, d = x.shape
    k = w.shape[1]
    return pl.pallas_call(
        _mm_body,
        grid=(n // bn,),
        in_specs=[
            pl.BlockSpec((bn, d), lambda i: (i, 0)),
            pl.BlockSpec((d, k), lambda i: (0, 0)),
        ],
        out_specs=pl.BlockSpec((bn, k), lambda i: (i, 0)),
        out_shape=jax.ShapeDtypeStruct((n, k), jnp.float32),
    )(x, w)


def _gat_layer(x, ei, W, a_src, a_dst, b, heads, out_ch, concat):
    n = x.shape[0]
    src, dst = ei[0], ei[1]
    h = _matmul(x, W).reshape(n, heads, out_ch)
    asrc = jnp.sum(h * a_src[None, :, :], axis=-1)
    adst = jnp.sum(h * a_dst[None, :, :], axis=-1)
    alpha = jax.nn.leaky_relu(asrc[src] + adst[dst], 0.2)
    gb = jax.nn.leaky_relu(jnp.max(asrc, axis=0) + jnp.max(adst, axis=0), 0.2)
    ealpha = jnp.exp(alpha - gb[None, :])
    denom = jax.ops.segment_sum(ealpha, dst, num_segments=n)
    attn = ealpha / (denom[dst] + 1e-16)
    out = jax.ops.segment_sum(h[src] * attn[:, :, None], dst, num_segments=n)
    if concat:
        out = out.reshape(n, heads * out_ch)
    else:
        out = out.mean(axis=1)
    return out + b


def kernel(x, edge_index, batch, W1, a_src1, a_dst1, b1, gamma1, beta1, W2, a_src2, a_dst2, b2, Wc1, bc1, Wc2, bc2):
    ar = jnp.arange(N, dtype=edge_index.dtype)
    ei = jnp.concatenate([edge_index, jnp.stack([ar, ar])], axis=1)
    h = _gat_layer(x, ei, W1, a_src1, a_dst1, b1, H, C, True)
    mu = h.mean(axis=0)
    var = h.var(axis=0)
    h = (h - mu) / jnp.sqrt(var + 1e-5) * gamma1 + beta1
    h = jax.nn.relu(h)
    h = _gat_layer(h, ei, W2, a_src2, a_dst2, b2, 1, C, False)
    h = jax.nn.relu(h)
    sums = jax.ops.segment_sum(h, batch, num_segments=B)
    counts = jax.ops.segment_sum(jnp.ones((h.shape[0], 1), dtype=h.dtype), batch, num_segments=B)
    g = sums / jnp.maximum(counts, 1.0)
    g = jax.nn.relu(g @ Wc1 + bc1)
    return g @ Wc2 + bc2


# speedup vs baseline: 5.0688x; 4.7244x over previous
import functools

import jax
import jax.numpy as jnp
from jax import lax
from jax.experimental import pallas as pl
from jax.experimental.pallas import tpu as pltpu
from jax.experimental.pallas import tpu_sc as plsc

N = 10000
D = 128
H = 8
C = 128
B = 64
OUT = 10

E_TOTAL = 330000
NPAD = 10240
PADN = N
EPAD = 360448
CH = 64
SUP = 16
NCH1 = EPAD // 16 // CH
NCH2 = EPAD // 32 // CH
G1 = NCH1 // SUP
G2 = NCH2 // SUP
NSL = NPAD // 16
NEG = -1e30
NF = float(N)


def _full16(v):
    return jnp.zeros((16,), jnp.int32) + v


def _leaky(t):
    return jnp.where(t >= 0.0, t, t * 0.2)


def _k1_body(x_ref, w_ref, o_ref):
    o_ref[0] = jnp.dot(x_ref[...], w_ref[0], preferred_element_type=jnp.float32)


def _k1(xp, w1r):
    return pl.pallas_call(
        _k1_body,
        grid=(NPAD // 512, H),
        in_specs=[
            pl.BlockSpec((512, D), lambda i, h: (i, 0)),
            pl.BlockSpec((1, D, C), lambda i, h: (h, 0, 0)),
        ],
        out_specs=pl.BlockSpec((1, 512, C), lambda i, h: (h, i, 0)),
        out_shape=jax.ShapeDtypeStruct((H, NPAD, C), jnp.float32),
    )(xp, w1r)


def _k1b_body(x_ref, w_ref, o_ref, gmax_ref):
    a = lax.dot_general(w_ref[...], x_ref[...], (((1,), (1,)), ((), ())),
                        preferred_element_type=jnp.float32)
    o_ref[...] = a
    m = jnp.broadcast_to(jnp.max(a, axis=1, keepdims=True), (16, 128))

    @pl.when(pl.program_id(0) == 0)
    def _():
        gmax_ref[...] = m

    @pl.when(pl.program_id(0) > 0)
    def _():
        gmax_ref[...] = jnp.maximum(gmax_ref[...], m)


def _k1b(xp, wsdT):
    return pl.pallas_call(
        _k1b_body,
        grid=(NPAD // 1280,),
        in_specs=[
            pl.BlockSpec((1280, D), lambda i: (i, 0)),
            pl.BlockSpec((16, D), lambda i: (0, 0)),
        ],
        out_specs=[
            pl.BlockSpec((16, 1280), lambda i: (0, i)),
            pl.BlockSpec((16, 128), lambda i: (0, 0)),
        ],
        out_shape=[
            jax.ShapeDtypeStruct((16, NPAD), jnp.float32),
            jax.ShapeDtypeStruct((16, 128), jnp.float32),
        ],
    )(xp, wsdT)


def _k4a_body(a_ref, s_ref, q_ref):
    blk = a_ref[...]
    s = jnp.sum(blk, axis=1)
    q = jnp.sum(blk * blk, axis=1)

    @pl.when(pl.program_id(0) == 0)
    def _():
        s_ref[...] = s
        q_ref[...] = q

    @pl.when(pl.program_id(0) > 0)
    def _():
        s_ref[...] += s
        q_ref[...] += q


def _k4a(acc1):
    return pl.pallas_call(
        _k4a_body,
        grid=(NPAD // 640,),
        in_specs=[pl.BlockSpec((H, 640, C), lambda i: (0, i, 0))],
        out_specs=[
            pl.BlockSpec((H, C), lambda i: (0, 0)),
            pl.BlockSpec((H, C), lambda i: (0, 0)),
        ],
        out_shape=[
            jax.ShapeDtypeStruct((H, C), jnp.float32),
            jax.ShapeDtypeStruct((H, C), jnp.float32),
        ],
    )(acc1)


def _k4b_body(acc_ref, s_ref, q_ref, g_ref, b_ref, w2_ref, a2_ref,
              h2_ref, asd2_ref, gmax2_ref):
    i = pl.program_id(0)
    h = pl.program_id(1)
    mu = s_ref[pl.ds(h, 1), :] / NF
    var = q_ref[pl.ds(h, 1), :] / NF - mu * mu
    inv = lax.rsqrt(var + 1e-5)
    gam = g_ref[pl.ds(h, 1), :]
    bet = b_ref[pl.ds(h, 1), :]
    hbn = jnp.maximum((acc_ref[0] - mu) * inv * gam + bet, 0.0)
    part = jnp.dot(hbn, w2_ref[0], preferred_element_type=jnp.float32)

    @pl.when(h == 0)
    def _():
        h2_ref[...] = part

    @pl.when(h > 0)
    def _():
        h2_ref[...] += part

    a2 = lax.dot_general(a2_ref[...], h2_ref[...], (((1,), (1,)), ((), ())),
                         preferred_element_type=jnp.float32)
    asd2_ref[...] = a2
    m = jnp.broadcast_to(jnp.max(a2, axis=1, keepdims=True), (16, 128))

    @pl.when(jnp.logical_and(h == H - 1, i == 0))
    def _():
        gmax2_ref[...] = m

    @pl.when(jnp.logical_and(h == H - 1, i > 0))
    def _():
        gmax2_ref[...] = jnp.maximum(gmax2_ref[...], m)


def _k4b(acc1, ssum, ssq, gam, bet, w2r, a2t):
    return pl.pallas_call(
        _k4b_body,
        grid=(NPAD // 1280, H),
        in_specs=[
            pl.BlockSpec((1, 1280, C), lambda i, h: (h, i, 0)),
            pl.BlockSpec((H, C), lambda i, h: (0, 0)),
            pl.BlockSpec((H, C), lambda i, h: (0, 0)),
            pl.BlockSpec((H, C), lambda i, h: (0, 0)),
            pl.BlockSpec((H, C), lambda i, h: (0, 0)),
            pl.BlockSpec((1, C, C), lambda i, h: (h, 0, 0)),
            pl.BlockSpec((16, C), lambda i, h: (0, 0)),
        ],
        out_specs=[
            pl.BlockSpec((1280, C), lambda i, h: (i, 0)),
            pl.BlockSpec((16, 1280), lambda i, h: (0, i)),
            pl.BlockSpec((16, 128), lambda i, h: (0, 0)),
        ],
        out_shape=[
            jax.ShapeDtypeStruct((NPAD, C), jnp.float32),
            jax.ShapeDtypeStruct((16, NPAD), jnp.float32),
            jax.ShapeDtypeStruct((16, 128), jnp.float32),
        ],
    )(acc1, ssum, ssq, gam, bet, w2r, a2t)


def _k7_body(p_ref, oh_ref, b2_ref, wc1_ref, bc1_ref, wc2_ref, bc2_ref,
             o_ref, sums_scr, cnt_scr):
    i = pl.program_id(0)
    hr = jnp.maximum(p_ref[0] + p_ref[1] + b2_ref[...][0:1, :], 0.0)
    oh = oh_ref[...]
    s = lax.dot_general(oh, hr, (((0,), (0,)), ((), ())),
                        preferred_element_type=jnp.float32)
    c = lax.dot_general(oh, jnp.ones_like(hr), (((0,), (0,)), ((), ())),
                        preferred_element_type=jnp.float32)

    @pl.when(i == 0)
    def _():
        sums_scr[...] = s
        cnt_scr[...] = c

    @pl.when(i > 0)
    def _():
        sums_scr[...] += s
        cnt_scr[...] += c

    @pl.when(i == NPAD // 1280 - 1)
    def _():
        g = sums_scr[...] / jnp.maximum(cnt_scr[...], 1.0)
        z1 = jnp.maximum(
            jnp.dot(g, wc1_ref[...], preferred_element_type=jnp.float32)
            + bc1_ref[...], 0.0)
        o_ref[...] = (jnp.dot(z1, wc2_ref[...],
                              preferred_element_type=jnp.float32)
                      + bc2_ref[...])


def _k7(acc2, oh, b2b, wc1p, bc1b, wc2p, bc2b):
    return pl.pallas_call(
        _k7_body,
        grid=(NPAD // 1280,),
        in_specs=[
            pl.BlockSpec((2, 1280, C), lambda i: (0, i, 0)),
            pl.BlockSpec((1280, 128), lambda i: (i, 0)),
            pl.BlockSpec((8, 128), lambda i: (0, 0)),
            pl.BlockSpec((128, 128), lambda i: (0, 0)),
            pl.BlockSpec((128, 128), lambda i: (0, 0)),
            pl.BlockSpec((128, 128), lambda i: (0, 0)),
            pl.BlockSpec((128, 128), lambda i: (0, 0)),
        ],
        out_specs=pl.BlockSpec((128, 128), lambda i: (0, 0)),
        out_shape=jax.ShapeDtypeStruct((128, 128), jnp.float32),
        scratch_shapes=[
            pltpu.VMEM((128, 128), jnp.float32),
            pltpu.VMEM((128, 128), jnp.float32),
        ],
    )(acc2, oh, b2b, wc1p, bc1b, wc2p, bc2b)


_MESH = dict(core_axis_name="c", subcore_axis_name="s", num_cores=2,
             num_subcores=16)


def _zero_rbuf(rbuf):
    def zr(r, _):
        for jj in range(8):
            rbuf[r, pl.ds(jj * 16, 16)] = jnp.zeros((16,), jnp.float32)
        return 0
    lax.fori_loop(0, CH, zr, 0)


def _k23_body(h1_hbm, asd_hbm, gmax_hbm, src_hbm, dst_hbm, acc_hbm,
              asrc_v, adst_v, den_v, g16_v, sbuf, dbuf, rbuf, sidx_v,
              attnb, ebuf, zbuf, sem, den_s, acc_s):
    c = lax.axis_index("c")
    s = lax.axis_index("s")

    def zb(j, _):
        zbuf[pl.ds(j * 16, 16)] = jnp.zeros((16,), jnp.float32)
        return 0
    lax.fori_loop(0, NSL // 16, zb, 0)

    def head_iter(hh, _):
        head = c * 4 + hh
        pltpu.sync_copy(asd_hbm.at[pl.ds(head * NPAD, NPAD)], asrc_v)
        pltpu.sync_copy(asd_hbm.at[pl.ds((head + 8) * NPAD, NPAD)], adst_v)
        asrc_v[pl.ds(PADN, 16)] = jnp.full((16,), NEG, jnp.float32)
        adst_v[pl.ds(PADN, 16)] = jnp.full((16,), NEG, jnp.float32)
        _zero_rbuf(rbuf)
        pltpu.sync_copy(zbuf, den_s.at[pl.ds(s * NSL, NSL)])
        for k in range(NSL // CH):
            pltpu.sync_copy(rbuf, acc_s.at[pl.ds(s * NSL + k * CH, CH)])
        plsc.subcore_barrier()

        pltpu.sync_copy(gmax_hbm.at[pl.ds(head * 128, 16)],
                        g16_v.at[pl.ds(0, 16)])
        pltpu.sync_copy(gmax_hbm.at[pl.ds((head + 8) * 128, 16)],
                        g16_v.at[pl.ds(16, 16)])
        gb = _leaky(g16_v[pl.ds(0, 16)] + g16_v[pl.ds(16, 16)])

        def g1(g, _):
            pltpu.sync_copy(src_hbm.at[s, pl.ds(g * SUP, SUP)], sbuf)
            pltpu.sync_copy(dst_hbm.at[s, pl.ds(g * SUP, SUP)], dbuf)

            def p1(ch, _):
                for j in range(CH // 16):
                    s16 = sbuf[ch, pl.ds(j * 16, 16)]
                    d16 = dbuf[ch, pl.ds(j * 16, 16)]
                    av = plsc.load_gather(asrc_v, [s16])
                    dv = plsc.load_gather(adst_v, [d16])
                    ebuf[pl.ds(j * 16, 16)] = jnp.exp(_leaky(av + dv) - gb)
                pltpu.sync_copy(ebuf, den_s.at[dbuf.at[ch]], add=True)
                return 0
            lax.fori_loop(0, SUP, p1, 0)
            return 0
        lax.fori_loop(0, G1, g1, 0)
        plsc.subcore_barrier()
        pltpu.sync_copy(den_s, den_v)
        hbase = head * NPAD

        def g3(g, _):
            pltpu.sync_copy(src_hbm.at[s, pl.ds(g * SUP, SUP)], sbuf)
            pltpu.sync_copy(dst_hbm.at[s, pl.ds(g * SUP, SUP)], dbuf)

            def p3(ch, _):
                for j in range(CH // 16):
                    s16 = sbuf[ch, pl.ds(j * 16, 16)]
                    d16 = dbuf[ch, pl.ds(j * 16, 16)]
                    av = plsc.load_gather(asrc_v, [s16])
                    dv = plsc.load_gather(adst_v, [d16])
                    ev = jnp.exp(_leaky(av + dv) - gb)
                    dd = plsc.load_gather(den_v, [d16])
                    attnb[pl.ds(j * 16, 16)] = ev / (dd + 1e-16)
                    sidx_v[pl.ds(j * 16, 16)] = s16 + hbase
                pltpu.async_copy(h1_hbm.at[sidx_v], rbuf, sem).wait()

                def rowfn(r, _):
                    bs = plsc.load_gather(attnb, [_full16(r)])
                    for jj in range(8):
                        rbuf[r, pl.ds(jj * 16, 16)] = (
                            rbuf[r, pl.ds(jj * 16, 16)] * bs)
                    return 0
                lax.fori_loop(0, CH, rowfn, 0)
                pltpu.sync_copy(rbuf, acc_s.at[dbuf.at[ch]], add=True)
                return 0
            lax.fori_loop(0, SUP, p3, 0)
            return 0
        lax.fori_loop(0, G1, g3, 0)
        plsc.subcore_barrier()
        pltpu.sync_copy(acc_s.at[pl.ds(s * NSL, NSL)],
                        acc_hbm.at[head, pl.ds(s * NSL, NSL)])
        plsc.subcore_barrier()
        return 0
    lax.fori_loop(0, 4, head_iter, 0)


def _sc_scratch(attn_extra):
    del attn_extra
    return [
        pltpu.VMEM((NPAD,), jnp.float32),
        pltpu.VMEM((NPAD,), jnp.float32),
        pltpu.VMEM((NPAD,), jnp.float32),
        pltpu.VMEM((32,), jnp.float32),
        pltpu.VMEM((SUP, CH), jnp.int32),
        pltpu.VMEM((SUP, CH), jnp.int32),
        pltpu.VMEM((CH, C), jnp.float32),
        pltpu.VMEM((CH,), jnp.int32),
        pltpu.VMEM((CH,), jnp.float32),
        pltpu.VMEM((CH,), jnp.float32),
        pltpu.VMEM((NSL,), jnp.float32),
        pltpu.SemaphoreType.DMA,
        pltpu.VMEM_SHARED((NPAD,), jnp.float32),
        pltpu.VMEM_SHARED((NPAD, C), jnp.float32),
    ]


def _k23(h1flat, asd1, gmax1, src16, dst16):
    f = functools.partial(
        pl.kernel,
        out_type=jax.ShapeDtypeStruct((H, NPAD, C), jnp.float32),
        mesh=plsc.VectorSubcoreMesh(**_MESH),
        compiler_params=pltpu.CompilerParams(needs_layout_passes=False),
        scratch_types=_sc_scratch(0),
    )(_k23_body)
    return f(h1flat, asd1, gmax1, src16, dst16)


def _k56_body(h2_hbm, asd_hbm, gmax_hbm, src16_hbm, dst16_hbm,
              src32_hbm, dst32_hbm, acc_hbm,
              asrc_v, adst_v, den_v, g16_v, sbuf, dbuf, rbuf, sidx_v,
              attnb, ebuf, zbuf, sem, den_s, acc_s):
    c = lax.axis_index("c")
    s = lax.axis_index("s")
    w = s * 2 + c
    pltpu.sync_copy(asd_hbm.at[pl.ds(0, NPAD)], asrc_v)
    pltpu.sync_copy(asd_hbm.at[pl.ds(NPAD, NPAD)], adst_v)
    asrc_v[pl.ds(PADN, 16)] = jnp.full((16,), NEG, jnp.float32)
    adst_v[pl.ds(PADN, 16)] = jnp.full((16,), NEG, jnp.float32)

    def zb(j, _):
        zbuf[pl.ds(j * 16, 16)] = jnp.zeros((16,), jnp.float32)
        return 0
    lax.fori_loop(0, NSL // 16, zb, 0)
    _zero_rbuf(rbuf)
    pltpu.sync_copy(zbuf, den_s.at[pl.ds(s * NSL, NSL)])
    for k in range(NSL // CH):
        pltpu.sync_copy(rbuf, acc_s.at[pl.ds(s * NSL + k * CH, CH)])
    plsc.subcore_barrier()

    pltpu.sync_copy(gmax_hbm.at[pl.ds(0, 16)], g16_v.at[pl.ds(0, 16)])
    pltpu.sync_copy(gmax_hbm.at[pl.ds(128, 16)], g16_v.at[pl.ds(16, 16)])
    gb = _leaky(g16_v[pl.ds(0, 16)] + g16_v[pl.ds(16, 16)])

    def g1(g, _):
        pltpu.sync_copy(src16_hbm.at[s, pl.ds(g * SUP, SUP)], sbuf)
        pltpu.sync_copy(dst16_hbm.at[s, pl.ds(g * SUP, SUP)], dbuf)

        def p1(ch, _):
            for j in range(CH // 16):
                s16 = sbuf[ch, pl.ds(j * 16, 16)]
                d16 = dbuf[ch, pl.ds(j * 16, 16)]
                av = plsc.load_gather(asrc_v, [s16])
                dv = plsc.load_gather(adst_v, [d16])
                ebuf[pl.ds(j * 16, 16)] = jnp.exp(_leaky(av + dv) - gb)
            pltpu.sync_copy(ebuf, den_s.at[dbuf.at[ch]], add=True)
            return 0
        lax.fori_loop(0, SUP, p1, 0)
        return 0
    lax.fori_loop(0, G1, g1, 0)
    plsc.subcore_barrier()
    pltpu.sync_copy(den_s, den_v)

    def g3(g, _):
        pltpu.sync_copy(src32_hbm.at[w, pl.ds(g * SUP, SUP)], sbuf)
        pltpu.sync_copy(dst32_hbm.at[w, pl.ds(g * SUP, SUP)], dbuf)

        def p3(ch, _):
            for j in range(CH // 16):
                s16 = sbuf[ch, pl.ds(j * 16, 16)]
                d16 = dbuf[ch, pl.ds(j * 16, 16)]
                av = plsc.load_gather(asrc_v, [s16])
                dv = plsc.load_gather(adst_v, [d16])
                ev = jnp.exp(_leaky(av + dv) - gb)
                dd = plsc.load_gather(den_v, [d16])
                attnb[pl.ds(j * 16, 16)] = ev / (dd + 1e-16)
                sidx_v[pl.ds(j * 16, 16)] = s16
            pltpu.async_copy(h2_hbm.at[sidx_v], rbuf, sem).wait()

            def rowfn(r, _):
                bs = plsc.load_gather(attnb, [_full16(r)])
                for jj in range(8):
                    rbuf[r, pl.ds(jj * 16, 16)] = (
                        rbuf[r, pl.ds(jj * 16, 16)] * bs)
                return 0
            lax.fori_loop(0, CH, rowfn, 0)
            pltpu.sync_copy(rbuf, acc_s.at[dbuf.at[ch]], add=True)
            return 0
        lax.fori_loop(0, SUP, p3, 0)
        return 0
    lax.fori_loop(0, G2, g3, 0)
    plsc.subcore_barrier()
    pltpu.sync_copy(acc_s.at[pl.ds(s * NSL, NSL)],
                    acc_hbm.at[c, pl.ds(s * NSL, NSL)])


def _k56(h2, asd2, gmax2, src16, dst16, src32, dst32):
    f = functools.partial(
        pl.kernel,
        out_type=jax.ShapeDtypeStruct((2, NPAD, C), jnp.float32),
        mesh=plsc.VectorSubcoreMesh(**_MESH),
        compiler_params=pltpu.CompilerParams(needs_layout_passes=False),
        scratch_types=_sc_scratch(0),
    )(_k56_body)
    return f(h2, asd2, gmax2, src16, dst16, src32, dst32)


def kernel(x, edge_index, batch, W1, a_src1, a_dst1, b1, gamma1, beta1,
           W2, a_src2, a_dst2, b2, Wc1, bc1, Wc2, bc2):
    del b1
    ar = jnp.arange(N, dtype=jnp.int32)
    padi = jnp.full((EPAD - E_TOTAL,), PADN, jnp.int32)
    src = jnp.concatenate([edge_index[0].astype(jnp.int32), ar, padi])
    dst = jnp.concatenate([edge_index[1].astype(jnp.int32), ar, padi])
    src16 = src.reshape(16, NCH1, CH)
    dst16 = dst.reshape(16, NCH1, CH)
    src32 = src.reshape(32, NCH2, CH)
    dst32 = dst.reshape(32, NCH2, CH)

    xp = jnp.zeros((NPAD, D), jnp.float32).at[:N].set(x)
    w1r = W1.reshape(D, H, C).transpose(1, 0, 2)
    wsdT = jnp.concatenate([
        jnp.einsum("hdc,hc->hd", w1r, a_src1),
        jnp.einsum("hdc,hc->hd", w1r, a_dst1),
    ], axis=0)
    w2r = W2.reshape(H, C, C)
    a2t = jnp.zeros((16, C), jnp.float32).at[0].set(a_src2[0]).at[1].set(a_dst2[0])
    gam = gamma1.reshape(H, C)
    bet = beta1.reshape(H, C)

    h1T = _k1(xp, w1r)
    asd1, gmax1 = _k1b(xp, wsdT)
    acc1 = _k23(h1T.reshape(H * NPAD, C), asd1.reshape(16 * NPAD),
                gmax1.reshape(2048), src16, dst16)
    ssum, ssq = _k4a(acc1)
    h2, asd2, gmax2 = _k4b(acc1, ssum, ssq, gam, bet, w2r, a2t)
    acc2 = _k56(h2, asd2.reshape(16 * NPAD), gmax2.reshape(2048), src16,
                dst16, src32, dst32)

    bp = jnp.concatenate([batch.astype(jnp.int32),
                          jnp.full((NPAD - N,), B, jnp.int32)])
    oh = (bp[:, None] == jnp.arange(128, dtype=jnp.int32)[None, :]
          ).astype(jnp.float32)
    b2b = jnp.broadcast_to(b2, (8, C))
    wc1p = jnp.zeros((128, 128), jnp.float32).at[:, :C // 2].set(Wc1)
    bc1b = jnp.broadcast_to(jnp.pad(bc1, (0, 128 - C // 2)), (128, 128))
    wc2p = jnp.zeros((128, 128), jnp.float32).at[:C // 2, :OUT].set(Wc2)
    bc2b = jnp.broadcast_to(jnp.pad(bc2, (0, 128 - OUT)), (128, 128))
    logits = _k7(acc2, oh, b2b, wc1p, bc1b, wc2p, bc2b)
    return logits[:B, :OUT]
